# Initial kernel scaffold; baseline (speedup 1.0000x reference)
import jax
import jax.numpy as jnp
from jax.experimental import pallas as pl


def kernel(edge_attr, W0, W1, W2):
    raise NotImplementedError("write your pallas kernel here")


# speedup vs baseline: 3.2753x; 3.2753x over previous
import functools

import jax
import jax.numpy as jnp
from jax import lax
from jax.experimental import pallas as pl
from jax.experimental.pallas import tpu as pltpu
from jax.experimental.pallas import tpu_sc as plsc

_SEG_DIMS = [5, 6, 2]
_EMB_DIM = 64
_E = 800000

_NC = 2
_NS = 16
_NW = _NC * _NS
_CHUNK = 128
_NCHUNKS = _E // _CHUNK
_GROUPS = _CHUNK // 16


def _seg_argmax(cols):
    best = cols[0]
    bidx = jnp.zeros((16,), jnp.int32)
    for j in range(1, len(cols)):
        m = cols[j] > best
        bidx = jnp.where(m, jnp.full((16,), j, jnp.int32), bidx)
        best = jnp.where(m, cols[j], best)
    return bidx


def _body(ea_hbm, t_hbm, out_hbm, ea_v, idx_v, rows_v, sem):
    wid = lax.axis_index("s") * _NC + lax.axis_index("c")
    n_my = (_NCHUNKS - wid + _NW - 1) // _NW

    def chunk_step(i, carry):
        chunk = wid + i * _NW
        base = chunk * _CHUNK
        pltpu.sync_copy(ea_hbm.at[pl.ds(base * 13, _CHUNK * 13)], ea_v)
        ramp13 = lax.iota(jnp.int32, 16) * 13
        for g in range(_GROUPS):
            flat = jnp.full((16,), g * 16 * 13, jnp.int32) + ramp13
            cols = [
                plsc.load_gather(ea_v, [flat + jnp.full((16,), j, jnp.int32)])
                for j in range(13)
            ]
            i0 = _seg_argmax(cols[0:5])
            i1 = _seg_argmax(cols[5:11])
            i2 = _seg_argmax(cols[11:13])
            idx_v[pl.ds(g * 16, 16)] = i0 * 12 + i1 * 2 + i2
        pltpu.async_copy(t_hbm.at[idx_v], rows_v, sem).wait()
        pltpu.sync_copy(rows_v, out_hbm.at[pl.ds(base, _CHUNK)])
        return carry

    lax.fori_loop(0, n_my, chunk_step, 0)


@jax.jit
def kernel(edge_attr, W0, W1, W2):
    table = (
        W0[:, None, None, :] + W1[None, :, None, :] + W2[None, None, :, :]
    ).reshape(_SEG_DIMS[0] * _SEG_DIMS[1] * _SEG_DIMS[2], _EMB_DIM)

    run = pl.kernel(
        _body,
        out_type=jax.ShapeDtypeStruct((_E, _EMB_DIM), jnp.float32),
        mesh=plsc.VectorSubcoreMesh(core_axis_name="c", subcore_axis_name="s"),
        scratch_types=[
            pltpu.VMEM((_CHUNK * 13,), jnp.float32),
            pltpu.VMEM((_CHUNK,), jnp.int32),
            pltpu.VMEM((_CHUNK, _EMB_DIM), jnp.float32),
            pltpu.SemaphoreType.DMA,
        ],
        compiler_params=pltpu.CompilerParams(
            needs_layout_passes=False, use_tc_tiling_on_sc=False
        ),
    )
    return run(edge_attr.reshape(-1), table)


# speedup vs baseline: 3.3263x; 1.0156x over previous
import jax
import jax.numpy as jnp
from jax import lax
from jax.experimental import pallas as pl
from jax.experimental.pallas import tpu as pltpu
from jax.experimental.pallas import tpu_sc as plsc

_SEG_DIMS = [5, 6, 2]
_EMB_DIM = 64
_E = 800000

_NC = 2
_NS = 16
_NW = _NC * _NS
_CHUNK = 128
_NCHUNKS = _E // _CHUNK
_GROUPS = _CHUNK // 16
_NBUF = 3


def _seg_argmax(cols):
    best = cols[0]
    bidx = jnp.zeros((16,), jnp.int32)
    for j in range(1, len(cols)):
        m = cols[j] > best
        bidx = jnp.where(m, jnp.full((16,), j, jnp.int32), bidx)
        best = jnp.where(m, cols[j], best)
    return bidx


def _body(ea_hbm, t_hbm, out_hbm, *scratch):
    ea_v = scratch[0:_NBUF]
    idx_v = scratch[_NBUF:2 * _NBUF]
    rows_v = scratch[2 * _NBUF:3 * _NBUF]
    in_sem = scratch[3 * _NBUF:4 * _NBUF]
    g_sem = scratch[4 * _NBUF:5 * _NBUF]
    o_sem = scratch[5 * _NBUF:6 * _NBUF]

    wid = lax.axis_index("s") * _NC + lax.axis_index("c")
    n_my = (_NCHUNKS - wid + _NW - 1) // _NW

    def chunk_of(j):
        return wid + j * _NW

    def start_in(b, j):
        base = chunk_of(j) * _CHUNK * 13
        pltpu.make_async_copy(
            ea_hbm.at[pl.ds(base, _CHUNK * 13)], ea_v[b], in_sem[b]
        ).start()

    def compute_idx(b):
        ramp13 = lax.iota(jnp.int32, 16) * 13
        for g in range(_GROUPS):
            flat = jnp.full((16,), g * 16 * 13, jnp.int32) + ramp13
            cols = [
                plsc.load_gather(ea_v[b], [flat + jnp.full((16,), j, jnp.int32)])
                for j in range(13)
            ]
            i0 = _seg_argmax(cols[0:5])
            i1 = _seg_argmax(cols[5:11])
            i2 = _seg_argmax(cols[11:13])
            idx_v[b][pl.ds(g * 16, 16)] = i0 * 12 + i1 * 2 + i2

    for b in range(_NBUF):
        @pl.when(b < n_my)
        def _(b=b):
            start_in(b, jnp.int32(b))

    n_iter = n_my + _NBUF
    n_outer = (_NCHUNKS // _NW + 1 + _NBUF + _NBUF - 1) // _NBUF

    def outer(o, carry):
        for b in range(_NBUF):
            j = o * _NBUF + b
            pb = (b - 1) % _NBUF

            @pl.when(jnp.logical_and(j >= _NBUF, j - _NBUF < n_my))
            def _():
                pltpu.make_async_copy(
                    rows_v[b],
                    out_hbm.at[pl.ds(0, _CHUNK)],
                    o_sem[b],
                ).wait()

            @pl.when(jnp.logical_and(j >= 1, j - 1 < n_my))
            def _():
                pltpu.make_async_copy(
                    t_hbm.at[idx_v[pb]], rows_v[pb], g_sem[pb]
                ).wait()
                pltpu.make_async_copy(
                    rows_v[pb],
                    out_hbm.at[pl.ds(chunk_of(j - 1) * _CHUNK, _CHUNK)],
                    o_sem[pb],
                ).start()

            @pl.when(j < n_my)
            def _():
                pltpu.make_async_copy(
                    ea_hbm.at[pl.ds(0, _CHUNK * 13)], ea_v[b], in_sem[b]
                ).wait()
                compute_idx(b)
                pltpu.make_async_copy(
                    t_hbm.at[idx_v[b]], rows_v[b], g_sem[b]
                ).start()

                @pl.when(j + _NBUF < n_my)
                def _():
                    start_in(b, j + _NBUF)

        return carry

    lax.fori_loop(0, n_outer, outer, jnp.int32(0), unroll=False)


@jax.jit
def kernel(edge_attr, W0, W1, W2):
    table = (
        W0[:, None, None, :] + W1[None, :, None, :] + W2[None, None, :, :]
    ).reshape(_SEG_DIMS[0] * _SEG_DIMS[1] * _SEG_DIMS[2], _EMB_DIM)

    scratch = (
        [pltpu.VMEM((_CHUNK * 13,), jnp.float32) for _ in range(_NBUF)]
        + [pltpu.VMEM((_CHUNK,), jnp.int32) for _ in range(_NBUF)]
        + [pltpu.VMEM((_CHUNK, _EMB_DIM), jnp.float32) for _ in range(_NBUF)]
        + [pltpu.SemaphoreType.DMA for _ in range(3 * _NBUF)]
    )
    run = pl.kernel(
        _body,
        out_type=jax.ShapeDtypeStruct((_E, _EMB_DIM), jnp.float32),
        mesh=plsc.VectorSubcoreMesh(core_axis_name="c", subcore_axis_name="s"),
        scratch_types=scratch,
        compiler_params=pltpu.CompilerParams(
            needs_layout_passes=False, use_tc_tiling_on_sc=False
        ),
    )
    return run(edge_attr.reshape(-1), table)


# speedup vs baseline: 4.8343x; 1.4534x over previous
import jax
import jax.numpy as jnp
from jax import lax
from jax.experimental import pallas as pl
from jax.experimental.pallas import tpu as pltpu
from jax.experimental.pallas import tpu_sc as plsc

_SEG_DIMS = [5, 6, 2]
_EMB_DIM = 64
_E = 800000

_NC = 2
_NS = 16
_NW = _NC * _NS
_CHUNK = 128
_NCHUNKS = _E // _CHUNK
_GROUPS = _CHUNK // 16
_NBUF = 3


def _seg_argmax(cols):
    best = cols[0]
    bidx = jnp.zeros((16,), jnp.int32)
    for j in range(1, len(cols)):
        m = cols[j] > best
        bidx = jnp.where(m, jnp.full((16,), j, jnp.int32), bidx)
        best = jnp.where(m, cols[j], best)
    return bidx


def _body(ea_hbm, t_hbm, out_hbm, *scratch):
    ea_v = scratch[0:_NBUF]
    idx_v = scratch[_NBUF:2 * _NBUF]
    rows_v = scratch[2 * _NBUF:3 * _NBUF]
    in_sem = scratch[3 * _NBUF:4 * _NBUF]
    g_sem = scratch[4 * _NBUF:5 * _NBUF]
    o_sem = scratch[5 * _NBUF:6 * _NBUF]

    wid = lax.axis_index("s") * _NC + lax.axis_index("c")
    n_my = (_NCHUNKS - wid + _NW - 1) // _NW

    def chunk_of(j):
        return wid + j * _NW

    def start_in(b, j):
        base = chunk_of(j) * _CHUNK * 13
        pltpu.make_async_copy(
            ea_hbm.at[pl.ds(base, _CHUNK * 13)], ea_v[b], in_sem[b]
        ).start()

    def compute_idx(b):
        rep_off = jnp.full((16,), 0, jnp.int32) + wid * 60
        ramp13 = lax.iota(jnp.int32, 16) * 13
        for g in range(_GROUPS):
            flat = jnp.full((16,), g * 16 * 13, jnp.int32) + ramp13
            cols = [
                plsc.load_gather(ea_v[b], [flat + jnp.full((16,), j, jnp.int32)])
                for j in range(13)
            ]
            i0 = _seg_argmax(cols[0:5])
            i1 = _seg_argmax(cols[5:11])
            i2 = _seg_argmax(cols[11:13])
            idx_v[b][pl.ds(g * 16, 16)] = i0 * 12 + i1 * 2 + i2 + rep_off

    for b in range(_NBUF):
        @pl.when(b < n_my)
        def _(b=b):
            start_in(b, jnp.int32(b))

    n_iter = n_my + _NBUF
    n_outer = (_NCHUNKS // _NW + 1 + _NBUF + _NBUF - 1) // _NBUF

    def outer(o, carry):
        for b in range(_NBUF):
            j = o * _NBUF + b
            pb = (b - 1) % _NBUF

            @pl.when(jnp.logical_and(j >= _NBUF, j - _NBUF < n_my))
            def _():
                pltpu.make_async_copy(
                    rows_v[b],
                    out_hbm.at[pl.ds(0, _CHUNK)],
                    o_sem[b],
                ).wait()

            @pl.when(jnp.logical_and(j >= 1, j - 1 < n_my))
            def _():
                pltpu.make_async_copy(
                    t_hbm.at[idx_v[pb]], rows_v[pb], g_sem[pb]
                ).wait()
                pltpu.make_async_copy(
                    rows_v[pb],
                    out_hbm.at[pl.ds(chunk_of(j - 1) * _CHUNK, _CHUNK)],
                    o_sem[pb],
                ).start()

            @pl.when(j < n_my)
            def _():
                pltpu.make_async_copy(
                    ea_hbm.at[pl.ds(0, _CHUNK * 13)], ea_v[b], in_sem[b]
                ).wait()
                compute_idx(b)
                pltpu.make_async_copy(
                    t_hbm.at[idx_v[b]], rows_v[b], g_sem[b]
                ).start()

                @pl.when(j + _NBUF < n_my)
                def _():
                    start_in(b, j + _NBUF)

        return carry

    lax.fori_loop(0, n_outer, outer, jnp.int32(0), unroll=False)


@jax.jit
def kernel(edge_attr, W0, W1, W2):
    table = (
        W0[:, None, None, :] + W1[None, :, None, :] + W2[None, None, :, :]
    ).reshape(_SEG_DIMS[0] * _SEG_DIMS[1] * _SEG_DIMS[2], _EMB_DIM)
    table = jnp.tile(table, (_NW, 1))

    scratch = (
        [pltpu.VMEM((_CHUNK * 13,), jnp.float32) for _ in range(_NBUF)]
        + [pltpu.VMEM((_CHUNK,), jnp.int32) for _ in range(_NBUF)]
        + [pltpu.VMEM((_CHUNK, _EMB_DIM), jnp.float32) for _ in range(_NBUF)]
        + [pltpu.SemaphoreType.DMA for _ in range(3 * _NBUF)]
    )
    run = pl.kernel(
        _body,
        out_type=jax.ShapeDtypeStruct((_E, _EMB_DIM), jnp.float32),
        mesh=plsc.VectorSubcoreMesh(core_axis_name="c", subcore_axis_name="s"),
        scratch_types=scratch,
        compiler_params=pltpu.CompilerParams(
            needs_layout_passes=False, use_tc_tiling_on_sc=False
        ),
    )
    return run(edge_attr.reshape(-1), table)
